# baseline (speedup 1.0000x reference)
import functools

import jax
import jax.numpy as jnp
from jax import lax
from jax.experimental import pallas as pl
from jax.experimental.pallas import tpu as pltpu
from jax.experimental.pallas import tpu_sc as plsc

N = 50000
E = 800000
H = 32
F = 64

NC = 2
NS = 16
NW = NC * NS
EPW = 25600
E_PAD = NW * EPW
CHUNK = 64
GROUP = 8
EPG = CHUNK * GROUP
NGROUPS = EPW // EPG
NACC = 50176
ACC_PT = NACC // NS
DUMP = N

BN = 2000
NBN = N // BN
BE = 4000
NBE = E // BE

_f32 = jnp.float32


def _dot(a, b):
    return jnp.dot(a, b, preferred_element_type=_f32)


def _enc_body(x_ref, w_ref, b_ref, o_ref):
    o_ref[...] = jnp.maximum(_dot(x_ref[...], w_ref[...]) + b_ref[...], 0.0)


def _enc_nodes(x2, w, b):
    m = x2.shape[0]
    return pl.pallas_call(
        _enc_body,
        grid=(m // BN,),
        in_specs=[pl.BlockSpec((BN, F), lambda i: (i, 0)),
                  pl.BlockSpec((F, H), lambda i: (0, 0)),
                  pl.BlockSpec((1, H), lambda i: (0, 0))],
        out_specs=pl.BlockSpec((BN, H), lambda i: (i, 0)),
        out_shape=jax.ShapeDtypeStruct((m, H), _f32),
    )(x2, w, b)


def _glob_enc(g, w, b):
    return pl.pallas_call(
        _enc_body,
        out_shape=jax.ShapeDtypeStruct((2, H), _f32),
    )(g, w, b)


def _prep_body(ex_ref, hx_ref, hg_ref, ws1, ws2, wr1, wr2, ag, beb, tw_ref):
    ce = _dot(hg_ref[...], ag[...]) + beb[...]
    s = _dot(ex_ref[...], ws1[...]) + _dot(hx_ref[...], ws2[...]) + ce
    r = _dot(ex_ref[...], wr1[...]) + _dot(hx_ref[...], wr2[...])
    tw_ref[...] = jnp.concatenate([s, r, s, r], axis=1)


def _prep(ex, hx, hg, ws1, ws2, wr1, wr2, ag, beb):
    wspec = pl.BlockSpec((H, H), lambda i: (0, 0))
    return pl.pallas_call(
        _prep_body,
        grid=(NBN,),
        in_specs=[pl.BlockSpec((BN, H), lambda i: (i, 0)),
                  pl.BlockSpec((BN, H), lambda i: (i, 0)),
                  pl.BlockSpec((1, H), lambda i: (0, 0)),
                  wspec, wspec, wspec, wspec, wspec,
                  pl.BlockSpec((1, H), lambda i: (0, 0))],
        out_specs=pl.BlockSpec((BN, 128), lambda i: (i, 0)),
        out_shape=jax.ShapeDtypeStruct((N, 128), _f32),
    )(ex, hx, hg, ws1, ws2, wr1, wr2, ag, beb)


def _z0_body(ea_ref, wenc, benc, a1s, z_ref):
    e = jnp.maximum(ea_ref[...] * wenc[...] + benc[...], 0.0)
    z_ref[...] = _dot(e, a1s[...])


def _z_step0(ea, wenc, benc, a1s):
    return pl.pallas_call(
        _z0_body,
        grid=(NBE,),
        in_specs=[pl.BlockSpec((BE, 1), lambda i: (i, 0)),
                  pl.BlockSpec((1, H), lambda i: (0, 0)),
                  pl.BlockSpec((1, H), lambda i: (0, 0)),
                  pl.BlockSpec((H, H), lambda i: (0, 0))],
        out_specs=pl.BlockSpec((BE, H), lambda i: (i, 0)),
        out_shape=jax.ShapeDtypeStruct((E_PAD, H), _f32),
    )(ea, wenc, benc, a1s)


def _z1_body(ea_ref, he_ref, wenc, benc, a1, a2, z_ref):
    e = jnp.maximum(ea_ref[...] * wenc[...] + benc[...], 0.0)
    z_ref[...] = _dot(e, a1[...]) + _dot(he_ref[...], a2[...])


def _z_step1(ea, he, wenc, benc, a1, a2):
    return pl.pallas_call(
        _z1_body,
        grid=(NBE,),
        in_specs=[pl.BlockSpec((BE, 1), lambda i: (i, 0)),
                  pl.BlockSpec((BE, H), lambda i: (i, 0)),
                  pl.BlockSpec((1, H), lambda i: (0, 0)),
                  pl.BlockSpec((1, H), lambda i: (0, 0)),
                  pl.BlockSpec((H, H), lambda i: (0, 0)),
                  pl.BlockSpec((H, H), lambda i: (0, 0))],
        out_specs=pl.BlockSpec((BE, H), lambda i: (i, 0)),
        out_shape=jax.ShapeDtypeStruct((E_PAD, H), _f32),
    )(ea, he, wenc, benc, a1, a2)


def _node_body(ex_ref, hx_ref, rp_ref, sp_ref, hg_ref,
               wn1, wn2, wn3, wn4, wn5, bnb,
               g1, g2, g3, bgb, wd1, bd1, wd2, bd2,
               nx_ref, dec_ref, ng_ref, sx_acc, se_acc):
    i = pl.program_id(0)
    ra = rp_ref[0, :, 0:H] + rp_ref[1, :, 0:H]
    sa = sp_ref[0, :, 0:H] + sp_ref[1, :, 0:H]
    cn = _dot(hg_ref[...], wn5[...]) + bnb[...]
    nx = jnp.maximum(_dot(ex_ref[...], wn1[...]) + _dot(hx_ref[...], wn2[...])
                     + _dot(ra, wn3[...]) + _dot(sa, wn4[...]) + cn, 0.0)
    nx_ref[...] = nx
    d1 = jnp.maximum(_dot(nx, wd1[...]) + bd1[...], 0.0)
    dec_ref[...] = _dot(d1, wd2[...]) + bd2[...]
    sx = jnp.sum(nx, axis=0, keepdims=True)
    se = jnp.sum(ra, axis=0, keepdims=True)

    @pl.when(i == 0)
    def _():
        sx_acc[...] = sx
        se_acc[...] = se

    @pl.when(i > 0)
    def _():
        sx_acc[...] = sx_acc[...] + sx
        se_acc[...] = se_acc[...] + se

    @pl.when(i == NBN - 1)
    def _():
        mx = sx_acc[...] * (1.0 / N)
        me = se_acc[...] * (1.0 / E)
        ng_ref[...] = jnp.maximum(_dot(mx, g1[...]) + _dot(me, g2[...])
                                  + _dot(hg_ref[...], g3[...]) + bgb[...], 0.0)


def _node_block(ex, hx, rp, sp, hg, wn1, wn2, wn3, wn4, wn5, bnb,
                g1, g2, g3, bgb, wd1, bd1, wd2, bd2):
    wspec = pl.BlockSpec((H, H), lambda i: (0, 0))
    bspec = pl.BlockSpec((1, H), lambda i: (0, 0))
    return pl.pallas_call(
        _node_body,
        grid=(NBN,),
        in_specs=[pl.BlockSpec((BN, H), lambda i: (i, 0)),
                  pl.BlockSpec((BN, H), lambda i: (i, 0)),
                  pl.BlockSpec((2, BN, 128), lambda i: (0, i, 0)),
                  pl.BlockSpec((2, BN, 128), lambda i: (0, i, 0)),
                  bspec,
                  wspec, wspec, wspec, wspec, wspec, bspec,
                  wspec, wspec, wspec, bspec,
                  wspec, bspec,
                  pl.BlockSpec((H, 1), lambda i: (0, 0)),
                  pl.BlockSpec((1, 1), lambda i: (0, 0))],
        out_specs=[pl.BlockSpec((BN, H), lambda i: (i, 0)),
                   pl.BlockSpec((BN, 1), lambda i: (i, 0)),
                   pl.BlockSpec((1, H), lambda i: (0, 0))],
        out_shape=[jax.ShapeDtypeStruct((N, H), _f32),
                   jax.ShapeDtypeStruct((N, 1), _f32),
                   jax.ShapeDtypeStruct((1, H), _f32)],
        scratch_shapes=[pltpu.VMEM((1, H), _f32),
                        pltpu.VMEM((1, H), _f32)],
    )(ex, hx, rp, sp, hg, wn1, wn2, wn3, wn4, wn5, bnb,
      g1, g2, g3, bgb, wd1, bd1, wd2, bd2)


_SC_MESH = plsc.VectorSubcoreMesh(core_axis_name="c", subcore_axis_name="s")

NSUB = 4
NPP = 12544
ACCROWS = 12800
ACC_PT2 = ACCROWS // NS
CPT = NPP // NS
BIGIDX = 1 << 26
NPART = NSUB * NPP


def _sc_ids():
    cid = lax.axis_index("c")
    sid = lax.axis_index("s")
    return cid, sid, sid * NC + cid


@functools.partial(
    pl.kernel,
    mesh=_SC_MESH,
    out_type=jax.ShapeDtypeStruct((E_PAD, H), _f32),
    scratch_types=[pltpu.VMEM((CHUNK, H), _f32),
                   pltpu.VMEM((CHUNK, 128), _f32),
                   pltpu.VMEM((CHUNK, 128), _f32),
                   pltpu.VMEM((CHUNK,), jnp.int32),
                   pltpu.VMEM((CHUNK,), jnp.int32),
                   pltpu.SemaphoreType.DMA,
                   pltpu.SemaphoreType.DMA],
)
def _sc_compute(z_hbm, tw_hbm, sidxg_hbm, ridxg_hbm, newe_hbm,
                z_v, sg_v, rg_v, ia_v, ib_v, sem1, sem2):
    cid, sid, wid = _sc_ids()

    @pl.loop(0, EPW // CHUNK)
    def _chunk(k):
        e0 = pl.multiple_of(wid * EPW + k * CHUNK, 8)
        pltpu.sync_copy(sidxg_hbm.at[pl.ds(e0, CHUNK)], ia_v)
        pltpu.sync_copy(ridxg_hbm.at[pl.ds(e0, CHUNK)], ib_v)
        cp1 = pltpu.async_copy(tw_hbm.at[ia_v], sg_v, sem1)
        cp2 = pltpu.async_copy(tw_hbm.at[ib_v], rg_v, sem2)
        pltpu.sync_copy(z_hbm.at[pl.ds(e0, CHUNK)], z_v)
        cp1.wait()
        cp2.wait()

        def _fuse(r, carry):
            v0 = (z_v[r, pl.ds(0, 16)] + sg_v[r, pl.ds(0, 16)]
                  + rg_v[r, pl.ds(32, 16)])
            z_v[r, pl.ds(0, 16)] = jnp.maximum(v0, 0.0)
            v1 = (z_v[r, pl.ds(16, 16)] + sg_v[r, pl.ds(16, 16)]
                  + rg_v[r, pl.ds(48, 16)])
            z_v[r, pl.ds(16, 16)] = jnp.maximum(v1, 0.0)
            return carry

        lax.fori_loop(0, CHUNK, _fuse, 0, unroll=8)
        pltpu.sync_copy(z_v, newe_hbm.at[pl.ds(e0, CHUNK)])


@functools.partial(
    pl.kernel,
    mesh=_SC_MESH,
    out_type=jax.ShapeDtypeStruct((2, NPART, 128), _f32),
    scratch_types=[pltpu.VMEM((32, 128), _f32),
                   pltpu.VMEM((CHUNK, 128), _f32),
                   pltpu.VMEM((CHUNK, H), _f32),
                   pltpu.VMEM((CHUNK,), jnp.int32),
                   pltpu.VMEM_SHARED((ACCROWS, 128), _f32)],
)
def _sc_aggregate(newe_hbm, idx_hbm, part_hbm, zb_v, wide_v, nb_v, ia_v, acc):
    cid, sid, wid = _sc_ids()
    zeros16 = jnp.zeros((16,), _f32)

    def _zero_rows(buf, nrows):
        def _z(r, carry):
            for h in range(8):
                buf[r, pl.ds(h * 16, 16)] = zeros16
            return carry

        lax.fori_loop(0, nrows, _z, 0, unroll=4)

    _zero_rows(zb_v, 32)
    _zero_rows(wide_v, CHUNK)

    for p in range(NSUB):
        base = p * NPP

        @pl.loop(0, ACC_PT2 // 32)
        def _zero_acc(k):
            pltpu.sync_copy(zb_v, acc.at[pl.ds(sid * ACC_PT2 + k * 32, 32)])

        plsc.subcore_barrier()

        @pl.loop(0, EPW // CHUNK)
        def _chunk(k):
            e0 = pl.multiple_of(wid * EPW + k * CHUNK, 8)
            pltpu.sync_copy(idx_hbm.at[pl.ds(e0, CHUNK)], ia_v)
            pltpu.sync_copy(newe_hbm.at[pl.ds(e0, CHUNK)], nb_v)

            def _st(r, carry):
                wide_v[r, pl.ds(0, 16)] = nb_v[r, pl.ds(0, 16)]
                wide_v[r, pl.ds(16, 16)] = nb_v[r, pl.ds(16, 16)]
                return carry

            lax.fori_loop(0, CHUNK, _st, 0, unroll=8)
            for g in range(CHUNK // 16):
                raw = ia_v[pl.ds(g * 16, 16)]
                t = raw - base
                ok = (t >= 0) & (t < NPP)
                ia_v[pl.ds(g * 16, 16)] = jnp.where(ok, t, NPP)
            pltpu.sync_copy(wide_v, acc.at[ia_v], add=True)

        plsc.subcore_barrier()

        @pl.loop(0, CPT // 16)
        def _co(k):
            off = pl.multiple_of(sid * CPT + k * 16, 8)
            pltpu.sync_copy(acc.at[pl.ds(off, 16)], wide_v.at[pl.ds(0, 16)])
            pltpu.sync_copy(wide_v.at[pl.ds(0, 16)],
                            part_hbm.at[cid, pl.ds(base + off, 16)])

        plsc.subcore_barrier()


def kernel(node_attr, edge_index, edge_attr, global_attr, x_masks, x_holdouts,
           indicates, stage, num_processing_steps,
           W_node_enc, b_node_enc, W_edge_enc, b_edge_enc, W_glob_enc,
           b_glob_enc, W_eb, b_eb, W_nb, b_nb, W_gb, b_gb,
           W_dec1, b_dec1, W_dec2, b_dec2):
    senders = edge_index[0]
    receivers = edge_index[1]
    pad = E_PAD - E
    zpad = jnp.zeros((pad,), jnp.int32)
    dpad = jnp.full((pad,), BIGIDX, jnp.int32)
    sidx_g = jnp.concatenate([senders, zpad])
    ridx_g = jnp.concatenate([receivers, zpad])
    sidx_s = jnp.concatenate([senders, dpad])
    ridx_s = jnp.concatenate([receivers, dpad])

    A1, A2 = W_eb[0:32], W_eb[32:64]
    Ws1, Ws2 = W_eb[64:96], W_eb[96:128]
    Wr1, Wr2 = W_eb[128:160], W_eb[160:192]
    Ag = W_eb[192:224]
    A1s = A1 + A2
    Wn1, Wn2, Wn3, Wn4, Wn5 = (W_nb[0:32], W_nb[32:64], W_nb[64:96],
                               W_nb[96:128], W_nb[128:160])
    G1, G2, G3 = W_gb[0:32], W_gb[32:64], W_gb[64:96]
    beb = b_eb.reshape(1, H)
    benc = b_edge_enc.reshape(1, H)
    bnb = b_nb.reshape(1, H)
    bgb = b_gb.reshape(1, H)
    bd1 = b_dec1.reshape(1, H)
    bd2 = b_dec2.reshape(1, 1)
    wd2 = W_dec2

    x2 = node_attr.reshape(2 * N, F)
    encx = _enc_nodes(x2, W_node_enc, b_node_enc.reshape(1, H))
    ex0, ex1 = encx[0:N], encx[N:2 * N]
    encg = _glob_enc(global_attr, W_glob_enc, b_glob_enc.reshape(1, H))
    hg0 = encg[0:1]

    ea0, ea1 = edge_attr[0], edge_attr[1]

    TW0 = _prep(ex0, ex0, hg0, Ws1, Ws2, Wr1, Wr2, Ag, beb)
    Z0 = _z_step0(ea0, W_edge_enc, benc, A1s)
    newe0 = _sc_compute(Z0, TW0, sidx_g, ridx_g)
    rp0 = _sc_aggregate(newe0, ridx_s)
    sp0 = _sc_aggregate(newe0, sidx_s)
    nx0, dec0, ng0 = _node_block(ex0, ex0, rp0, sp0, hg0,
                                 Wn1, Wn2, Wn3, Wn4, Wn5, bnb,
                                 G1, G2, G3, bgb, W_dec1, bd1, wd2, bd2)

    TW1 = _prep(ex1, nx0, ng0, Ws1, Ws2, Wr1, Wr2, Ag, beb)
    Z1 = _z_step1(ea1, newe0, W_edge_enc, benc, A1, A2)
    newe1 = _sc_compute(Z1, TW1, sidx_g, ridx_g)
    rp1 = _sc_aggregate(newe1, ridx_s)
    sp1 = _sc_aggregate(newe1, sidx_s)
    nx1, dec1, ng1 = _node_block(ex1, nx0, rp1, sp1, ng0,
                                 Wn1, Wn2, Wn3, Wn4, Wn5, bnb,
                                 G1, G2, G3, bgb, W_dec1, bd1, wd2, bd2)

    return jnp.stack([dec0, dec1])


# speedup vs baseline: 1.0412x; 1.0412x over previous
import functools

import jax
import jax.numpy as jnp
from jax import lax
from jax.experimental import pallas as pl
from jax.experimental.pallas import tpu as pltpu
from jax.experimental.pallas import tpu_sc as plsc

N = 50000
E = 800000
H = 32
F = 64

NC = 2
NS = 16
NW = NC * NS
EPW = 25600
E_PAD = NW * EPW
CHUNK = 128

BN = 2000
NBN = N // BN
BE = 4000
NBE = E // BE

_f32 = jnp.float32


def _dot(a, b):
    return jnp.dot(a, b, preferred_element_type=_f32)


def _enc_body(x_ref, w_ref, b_ref, o_ref):
    o_ref[...] = jnp.maximum(_dot(x_ref[...], w_ref[...]) + b_ref[...], 0.0)


def _enc_nodes(x2, w, b):
    m = x2.shape[0]
    return pl.pallas_call(
        _enc_body,
        grid=(m // BN,),
        in_specs=[pl.BlockSpec((BN, F), lambda i: (i, 0)),
                  pl.BlockSpec((F, H), lambda i: (0, 0)),
                  pl.BlockSpec((1, H), lambda i: (0, 0))],
        out_specs=pl.BlockSpec((BN, H), lambda i: (i, 0)),
        out_shape=jax.ShapeDtypeStruct((m, H), _f32),
    )(x2, w, b)


def _glob_enc(g, w, b):
    return pl.pallas_call(
        _enc_body,
        out_shape=jax.ShapeDtypeStruct((2, H), _f32),
    )(g, w, b)


def _prep_body(ex_ref, hx_ref, hg_ref, ws1, ws2, wr1, wr2, ag, beb, tw_ref):
    ce = _dot(hg_ref[...], ag[...]) + beb[...]
    s = _dot(ex_ref[...], ws1[...]) + _dot(hx_ref[...], ws2[...]) + ce
    r = _dot(ex_ref[...], wr1[...]) + _dot(hx_ref[...], wr2[...])
    tw_ref[...] = jnp.concatenate([s, r, s, r], axis=1)


def _prep(ex, hx, hg, ws1, ws2, wr1, wr2, ag, beb):
    wspec = pl.BlockSpec((H, H), lambda i: (0, 0))
    return pl.pallas_call(
        _prep_body,
        grid=(NBN,),
        in_specs=[pl.BlockSpec((BN, H), lambda i: (i, 0)),
                  pl.BlockSpec((BN, H), lambda i: (i, 0)),
                  pl.BlockSpec((1, H), lambda i: (0, 0)),
                  wspec, wspec, wspec, wspec, wspec,
                  pl.BlockSpec((1, H), lambda i: (0, 0))],
        out_specs=pl.BlockSpec((BN, 128), lambda i: (i, 0)),
        out_shape=jax.ShapeDtypeStruct((N, 128), _f32),
    )(ex, hx, hg, ws1, ws2, wr1, wr2, ag, beb)


def _z0_body(ea_ref, wenc, benc, a1s, z_ref):
    e = jnp.maximum(ea_ref[...] * wenc[...] + benc[...], 0.0)
    z_ref[...] = _dot(e, a1s[...])


def _z_step0(ea, wenc, benc, a1s):
    return pl.pallas_call(
        _z0_body,
        grid=(NBE,),
        in_specs=[pl.BlockSpec((BE, 1), lambda i: (i, 0)),
                  pl.BlockSpec((1, H), lambda i: (0, 0)),
                  pl.BlockSpec((1, H), lambda i: (0, 0)),
                  pl.BlockSpec((H, H), lambda i: (0, 0))],
        out_specs=pl.BlockSpec((BE, H), lambda i: (i, 0)),
        out_shape=jax.ShapeDtypeStruct((E_PAD, H), _f32),
    )(ea, wenc, benc, a1s)


def _z1_body(ea_ref, he_ref, wenc, benc, a1, a2, z_ref):
    e = jnp.maximum(ea_ref[...] * wenc[...] + benc[...], 0.0)
    z_ref[...] = _dot(e, a1[...]) + _dot(he_ref[...], a2[...])


def _z_step1(ea, he, wenc, benc, a1, a2):
    return pl.pallas_call(
        _z1_body,
        grid=(NBE,),
        in_specs=[pl.BlockSpec((BE, 1), lambda i: (i, 0)),
                  pl.BlockSpec((BE, H), lambda i: (i, 0)),
                  pl.BlockSpec((1, H), lambda i: (0, 0)),
                  pl.BlockSpec((1, H), lambda i: (0, 0)),
                  pl.BlockSpec((H, H), lambda i: (0, 0)),
                  pl.BlockSpec((H, H), lambda i: (0, 0))],
        out_specs=pl.BlockSpec((BE, H), lambda i: (i, 0)),
        out_shape=jax.ShapeDtypeStruct((E_PAD, H), _f32),
    )(ea, he, wenc, benc, a1, a2)


def _node_body(ex_ref, hx_ref, rp_ref, sp_ref, hg_ref,
               wn1, wn2, wn3, wn4, wn5, bnb,
               g1, g2, g3, bgb, wd1, bd1, wd2, bd2,
               nx_ref, dec_ref, ng_ref, sx_acc, se_acc):
    i = pl.program_id(0)
    ra = rp_ref[0, :, 0:H] + rp_ref[1, :, 0:H]
    sa = sp_ref[0, :, 0:H] + sp_ref[1, :, 0:H]
    cn = _dot(hg_ref[...], wn5[...]) + bnb[...]
    nx = jnp.maximum(_dot(ex_ref[...], wn1[...]) + _dot(hx_ref[...], wn2[...])
                     + _dot(ra, wn3[...]) + _dot(sa, wn4[...]) + cn, 0.0)
    nx_ref[...] = nx
    d1 = jnp.maximum(_dot(nx, wd1[...]) + bd1[...], 0.0)
    dec_ref[...] = _dot(d1, wd2[...]) + bd2[...]
    sx = jnp.sum(nx, axis=0, keepdims=True)
    se = jnp.sum(ra, axis=0, keepdims=True)

    @pl.when(i == 0)
    def _():
        sx_acc[...] = sx
        se_acc[...] = se

    @pl.when(i > 0)
    def _():
        sx_acc[...] = sx_acc[...] + sx
        se_acc[...] = se_acc[...] + se

    @pl.when(i == NBN - 1)
    def _():
        mx = sx_acc[...] * (1.0 / N)
        me = se_acc[...] * (1.0 / E)
        ng_ref[...] = jnp.maximum(_dot(mx, g1[...]) + _dot(me, g2[...])
                                  + _dot(hg_ref[...], g3[...]) + bgb[...], 0.0)


def _node_block(ex, hx, rp, sp, hg, wn1, wn2, wn3, wn4, wn5, bnb,
                g1, g2, g3, bgb, wd1, bd1, wd2, bd2):
    wspec = pl.BlockSpec((H, H), lambda i: (0, 0))
    bspec = pl.BlockSpec((1, H), lambda i: (0, 0))
    return pl.pallas_call(
        _node_body,
        grid=(NBN,),
        in_specs=[pl.BlockSpec((BN, H), lambda i: (i, 0)),
                  pl.BlockSpec((BN, H), lambda i: (i, 0)),
                  pl.BlockSpec((2, BN, 128), lambda i: (0, i, 0)),
                  pl.BlockSpec((2, BN, 128), lambda i: (0, i, 0)),
                  bspec,
                  wspec, wspec, wspec, wspec, wspec, bspec,
                  wspec, wspec, wspec, bspec,
                  wspec, bspec,
                  pl.BlockSpec((H, 1), lambda i: (0, 0)),
                  pl.BlockSpec((1, 1), lambda i: (0, 0))],
        out_specs=[pl.BlockSpec((BN, H), lambda i: (i, 0)),
                   pl.BlockSpec((BN, 1), lambda i: (i, 0)),
                   pl.BlockSpec((1, H), lambda i: (0, 0))],
        out_shape=[jax.ShapeDtypeStruct((N, H), _f32),
                   jax.ShapeDtypeStruct((N, 1), _f32),
                   jax.ShapeDtypeStruct((1, H), _f32)],
        scratch_shapes=[pltpu.VMEM((1, H), _f32),
                        pltpu.VMEM((1, H), _f32)],
    )(ex, hx, rp, sp, hg, wn1, wn2, wn3, wn4, wn5, bnb,
      g1, g2, g3, bgb, wd1, bd1, wd2, bd2)


_SC_MESH = plsc.VectorSubcoreMesh(core_axis_name="c", subcore_axis_name="s")

NSUB = 5
NPP = 11520
ACCROWS = 11776
ACC_PT2 = ACCROWS // NS
CPT = NPP // NS
BIGIDX = 1 << 26
NPART = NSUB * NPP


def _sc_ids():
    cid = lax.axis_index("c")
    sid = lax.axis_index("s")
    return cid, sid, sid * NC + cid


@functools.partial(
    pl.kernel,
    mesh=_SC_MESH,
    out_type=jax.ShapeDtypeStruct((E_PAD, H), _f32),
    scratch_types=[pltpu.VMEM((CHUNK, H), _f32),
                   pltpu.VMEM((CHUNK, 128), _f32),
                   pltpu.VMEM((CHUNK, 128), _f32),
                   pltpu.VMEM((CHUNK,), jnp.int32),
                   pltpu.VMEM((CHUNK,), jnp.int32),
                   pltpu.SemaphoreType.DMA,
                   pltpu.SemaphoreType.DMA,
                   pltpu.SemaphoreType.DMA,
                   pltpu.SemaphoreType.DMA,
                   pltpu.SemaphoreType.DMA],
)
def _sc_compute(z_hbm, tw_hbm, sidxg_hbm, ridxg_hbm, newe_hbm,
                z_v, sg_v, rg_v, ia_v, ib_v, sem1, sem2, sem3, sem4, sem5):
    cid, sid, wid = _sc_ids()

    @pl.loop(0, EPW // CHUNK)
    def _chunk(k):
        e0 = pl.multiple_of(wid * EPW + k * CHUNK, 8)
        ca = pltpu.async_copy(sidxg_hbm.at[pl.ds(e0, CHUNK)], ia_v, sem3)
        cb = pltpu.async_copy(ridxg_hbm.at[pl.ds(e0, CHUNK)], ib_v, sem4)
        cz = pltpu.async_copy(z_hbm.at[pl.ds(e0, CHUNK)], z_v, sem5)
        ca.wait()
        cb.wait()
        cp1 = pltpu.async_copy(tw_hbm.at[ia_v], sg_v, sem1)
        cp2 = pltpu.async_copy(tw_hbm.at[ib_v], rg_v, sem2)
        cz.wait()
        cp1.wait()
        cp2.wait()

        def _fuse(r, carry):
            v0 = (z_v[r, pl.ds(0, 16)] + sg_v[r, pl.ds(0, 16)]
                  + rg_v[r, pl.ds(32, 16)])
            z_v[r, pl.ds(0, 16)] = jnp.maximum(v0, 0.0)
            v1 = (z_v[r, pl.ds(16, 16)] + sg_v[r, pl.ds(16, 16)]
                  + rg_v[r, pl.ds(48, 16)])
            z_v[r, pl.ds(16, 16)] = jnp.maximum(v1, 0.0)
            return carry

        lax.fori_loop(0, CHUNK, _fuse, 0, unroll=8)
        pltpu.sync_copy(z_v, newe_hbm.at[pl.ds(e0, CHUNK)])


@functools.partial(
    pl.kernel,
    mesh=_SC_MESH,
    out_type=jax.ShapeDtypeStruct((2, NPART, 128), _f32),
    scratch_types=[pltpu.VMEM((16, 128), _f32),
                   pltpu.VMEM((CHUNK, 128), _f32),
                   pltpu.VMEM((CHUNK, H), _f32),
                   pltpu.VMEM((CHUNK,), jnp.int32),
                   pltpu.VMEM_SHARED((ACCROWS, 128), _f32),
                   pltpu.SemaphoreType.DMA,
                   pltpu.SemaphoreType.DMA],
)
def _sc_aggregate(newe_hbm, idx_hbm, part_hbm, zb_v, wide_v, nb_v, ia_v, acc,
                  sem1, sem2):
    cid, sid, wid = _sc_ids()
    zeros16 = jnp.zeros((16,), _f32)

    def _zero_rows(buf, nrows):
        def _z(r, carry):
            for h in range(8):
                buf[r, pl.ds(h * 16, 16)] = zeros16
            return carry

        lax.fori_loop(0, nrows, _z, 0, unroll=4)

    _zero_rows(zb_v, 16)
    _zero_rows(wide_v, CHUNK)

    for p in range(NSUB):
        base = p * NPP

        @pl.loop(0, ACC_PT2 // 16)
        def _zero_acc(k):
            pltpu.sync_copy(zb_v, acc.at[pl.ds(sid * ACC_PT2 + k * 16, 16)])

        plsc.subcore_barrier()

        @pl.loop(0, EPW // CHUNK)
        def _chunk(k):
            e0 = pl.multiple_of(wid * EPW + k * CHUNK, 8)
            ca = pltpu.async_copy(idx_hbm.at[pl.ds(e0, CHUNK)], ia_v, sem1)
            cb = pltpu.async_copy(newe_hbm.at[pl.ds(e0, CHUNK)], nb_v, sem2)
            ca.wait()
            cb.wait()

            def _st(r, carry):
                wide_v[r, pl.ds(0, 16)] = nb_v[r, pl.ds(0, 16)]
                wide_v[r, pl.ds(16, 16)] = nb_v[r, pl.ds(16, 16)]
                return carry

            lax.fori_loop(0, CHUNK, _st, 0, unroll=8)
            for g in range(CHUNK // 16):
                raw = ia_v[pl.ds(g * 16, 16)]
                t = raw - base
                ok = (t >= 0) & (t < NPP)
                ia_v[pl.ds(g * 16, 16)] = jnp.where(ok, t, NPP)
            pltpu.sync_copy(wide_v, acc.at[ia_v], add=True)

        plsc.subcore_barrier()

        @pl.loop(0, CPT // 16)
        def _co(k):
            off = pl.multiple_of(sid * CPT + k * 16, 8)
            pltpu.sync_copy(acc.at[pl.ds(off, 16)], wide_v.at[pl.ds(0, 16)])
            pltpu.sync_copy(wide_v.at[pl.ds(0, 16)],
                            part_hbm.at[cid, pl.ds(base + off, 16)])

        plsc.subcore_barrier()


def kernel(node_attr, edge_index, edge_attr, global_attr, x_masks, x_holdouts,
           indicates, stage, num_processing_steps,
           W_node_enc, b_node_enc, W_edge_enc, b_edge_enc, W_glob_enc,
           b_glob_enc, W_eb, b_eb, W_nb, b_nb, W_gb, b_gb,
           W_dec1, b_dec1, W_dec2, b_dec2):
    senders = edge_index[0]
    receivers = edge_index[1]
    pad = E_PAD - E
    zpad = jnp.zeros((pad,), jnp.int32)
    dpad = jnp.full((pad,), BIGIDX, jnp.int32)
    sidx_g = jnp.concatenate([senders, zpad])
    ridx_g = jnp.concatenate([receivers, zpad])
    sidx_s = jnp.concatenate([senders, dpad])
    ridx_s = jnp.concatenate([receivers, dpad])

    A1, A2 = W_eb[0:32], W_eb[32:64]
    Ws1, Ws2 = W_eb[64:96], W_eb[96:128]
    Wr1, Wr2 = W_eb[128:160], W_eb[160:192]
    Ag = W_eb[192:224]
    A1s = A1 + A2
    Wn1, Wn2, Wn3, Wn4, Wn5 = (W_nb[0:32], W_nb[32:64], W_nb[64:96],
                               W_nb[96:128], W_nb[128:160])
    G1, G2, G3 = W_gb[0:32], W_gb[32:64], W_gb[64:96]
    beb = b_eb.reshape(1, H)
    benc = b_edge_enc.reshape(1, H)
    bnb = b_nb.reshape(1, H)
    bgb = b_gb.reshape(1, H)
    bd1 = b_dec1.reshape(1, H)
    bd2 = b_dec2.reshape(1, 1)
    wd2 = W_dec2

    x2 = node_attr.reshape(2 * N, F)
    encx = _enc_nodes(x2, W_node_enc, b_node_enc.reshape(1, H))
    ex0, ex1 = encx[0:N], encx[N:2 * N]
    encg = _glob_enc(global_attr, W_glob_enc, b_glob_enc.reshape(1, H))
    hg0 = encg[0:1]

    ea0, ea1 = edge_attr[0], edge_attr[1]

    TW0 = _prep(ex0, ex0, hg0, Ws1, Ws2, Wr1, Wr2, Ag, beb)
    Z0 = _z_step0(ea0, W_edge_enc, benc, A1s)
    newe0 = _sc_compute(Z0, TW0, sidx_g, ridx_g)
    rp0 = _sc_aggregate(newe0, ridx_s)
    sp0 = _sc_aggregate(newe0, sidx_s)
    nx0, dec0, ng0 = _node_block(ex0, ex0, rp0, sp0, hg0,
                                 Wn1, Wn2, Wn3, Wn4, Wn5, bnb,
                                 G1, G2, G3, bgb, W_dec1, bd1, wd2, bd2)

    TW1 = _prep(ex1, nx0, ng0, Ws1, Ws2, Wr1, Wr2, Ag, beb)
    Z1 = _z_step1(ea1, newe0, W_edge_enc, benc, A1, A2)
    newe1 = _sc_compute(Z1, TW1, sidx_g, ridx_g)
    rp1 = _sc_aggregate(newe1, ridx_s)
    sp1 = _sc_aggregate(newe1, sidx_s)
    nx1, dec1, ng1 = _node_block(ex1, nx0, rp1, sp1, ng0,
                                 Wn1, Wn2, Wn3, Wn4, Wn5, bnb,
                                 G1, G2, G3, bgb, W_dec1, bd1, wd2, bd2)

    return jnp.stack([dec0, dec1])


# speedup vs baseline: 1.1138x; 1.0697x over previous
import functools

import jax
import jax.numpy as jnp
from jax import lax
from jax.experimental import pallas as pl
from jax.experimental.pallas import tpu as pltpu
from jax.experimental.pallas import tpu_sc as plsc

N = 50000
E = 800000
H = 32
F = 64

NC = 2
NS = 16
NW = NC * NS
EPW = 25600
E_PAD = NW * EPW
CHUNK = 128

BN = 2000
NBN = N // BN
BE = 4000
NBE = E // BE

_f32 = jnp.float32


def _dot(a, b):
    return jnp.dot(a, b, preferred_element_type=_f32)


def _enc_body(x_ref, w_ref, b_ref, o_ref):
    o_ref[...] = jnp.maximum(_dot(x_ref[...], w_ref[...]) + b_ref[...], 0.0)


def _enc_nodes(x2, w, b):
    m = x2.shape[0]
    return pl.pallas_call(
        _enc_body,
        grid=(m // BN,),
        in_specs=[pl.BlockSpec((BN, F), lambda i: (i, 0)),
                  pl.BlockSpec((F, H), lambda i: (0, 0)),
                  pl.BlockSpec((1, H), lambda i: (0, 0))],
        out_specs=pl.BlockSpec((BN, H), lambda i: (i, 0)),
        out_shape=jax.ShapeDtypeStruct((m, H), _f32),
    )(x2, w, b)


def _glob_enc(g, w, b):
    return pl.pallas_call(
        _enc_body,
        out_shape=jax.ShapeDtypeStruct((2, H), _f32),
    )(g, w, b)


def _prep_body(ex_ref, hx_ref, hg_ref, ws1, ws2, wr1, wr2, ag, beb, tw_ref):
    ce = _dot(hg_ref[...], ag[...]) + beb[...]
    s = _dot(ex_ref[...], ws1[...]) + _dot(hx_ref[...], ws2[...]) + ce
    r = _dot(ex_ref[...], wr1[...]) + _dot(hx_ref[...], wr2[...])
    tw_ref[...] = jnp.concatenate([s, r, s, r], axis=1)


def _prep(ex, hx, hg, ws1, ws2, wr1, wr2, ag, beb):
    wspec = pl.BlockSpec((H, H), lambda i: (0, 0))
    return pl.pallas_call(
        _prep_body,
        grid=(NBN,),
        in_specs=[pl.BlockSpec((BN, H), lambda i: (i, 0)),
                  pl.BlockSpec((BN, H), lambda i: (i, 0)),
                  pl.BlockSpec((1, H), lambda i: (0, 0)),
                  wspec, wspec, wspec, wspec, wspec,
                  pl.BlockSpec((1, H), lambda i: (0, 0))],
        out_specs=pl.BlockSpec((BN, 128), lambda i: (i, 0)),
        out_shape=jax.ShapeDtypeStruct((N, 128), _f32),
    )(ex, hx, hg, ws1, ws2, wr1, wr2, ag, beb)


def _z0_body(ea_ref, wenc, benc, a1s, z_ref):
    e = jnp.maximum(ea_ref[...] * wenc[...] + benc[...], 0.0)
    z_ref[...] = _dot(e, a1s[...])


def _z_step0(ea, wenc, benc, a1s):
    return pl.pallas_call(
        _z0_body,
        grid=(NBE,),
        in_specs=[pl.BlockSpec((BE, 1), lambda i: (i, 0)),
                  pl.BlockSpec((1, H), lambda i: (0, 0)),
                  pl.BlockSpec((1, H), lambda i: (0, 0)),
                  pl.BlockSpec((H, H), lambda i: (0, 0))],
        out_specs=pl.BlockSpec((BE, H), lambda i: (i, 0)),
        out_shape=jax.ShapeDtypeStruct((E_PAD, H), _f32),
    )(ea, wenc, benc, a1s)


def _z1_body(ea_ref, he_ref, wenc, benc, a1, a2, z_ref):
    e = jnp.maximum(ea_ref[...] * wenc[...] + benc[...], 0.0)
    z_ref[...] = _dot(e, a1[...]) + _dot(he_ref[...], a2[...])


def _z_step1(ea, he, wenc, benc, a1, a2):
    return pl.pallas_call(
        _z1_body,
        grid=(NBE,),
        in_specs=[pl.BlockSpec((BE, 1), lambda i: (i, 0)),
                  pl.BlockSpec((BE, H), lambda i: (i, 0)),
                  pl.BlockSpec((1, H), lambda i: (0, 0)),
                  pl.BlockSpec((1, H), lambda i: (0, 0)),
                  pl.BlockSpec((H, H), lambda i: (0, 0)),
                  pl.BlockSpec((H, H), lambda i: (0, 0))],
        out_specs=pl.BlockSpec((BE, H), lambda i: (i, 0)),
        out_shape=jax.ShapeDtypeStruct((E_PAD, H), _f32),
    )(ea, he, wenc, benc, a1, a2)


def _node_body(ex_ref, hx_ref, rp_ref, sp_ref, hg_ref,
               wn1, wn2, wn3, wn4, wn5, bnb,
               g1, g2, g3, bgb, wd1, bd1, wd2, bd2,
               nx_ref, dec_ref, ng_ref, sx_acc, se_acc):
    i = pl.program_id(0)
    ra = rp_ref[0, :, 0:H] + rp_ref[1, :, 0:H]
    sa = sp_ref[0, :, 0:H] + sp_ref[1, :, 0:H]
    cn = _dot(hg_ref[...], wn5[...]) + bnb[...]
    nx = jnp.maximum(_dot(ex_ref[...], wn1[...]) + _dot(hx_ref[...], wn2[...])
                     + _dot(ra, wn3[...]) + _dot(sa, wn4[...]) + cn, 0.0)
    nx_ref[...] = nx
    d1 = jnp.maximum(_dot(nx, wd1[...]) + bd1[...], 0.0)
    dec_ref[...] = _dot(d1, wd2[...]) + bd2[...]
    sx = jnp.sum(nx, axis=0, keepdims=True)
    se = jnp.sum(ra, axis=0, keepdims=True)

    @pl.when(i == 0)
    def _():
        sx_acc[...] = sx
        se_acc[...] = se

    @pl.when(i > 0)
    def _():
        sx_acc[...] = sx_acc[...] + sx
        se_acc[...] = se_acc[...] + se

    @pl.when(i == NBN - 1)
    def _():
        mx = sx_acc[...] * (1.0 / N)
        me = se_acc[...] * (1.0 / E)
        ng_ref[...] = jnp.maximum(_dot(mx, g1[...]) + _dot(me, g2[...])
                                  + _dot(hg_ref[...], g3[...]) + bgb[...], 0.0)


def _node_block(ex, hx, rp, sp, hg, wn1, wn2, wn3, wn4, wn5, bnb,
                g1, g2, g3, bgb, wd1, bd1, wd2, bd2):
    wspec = pl.BlockSpec((H, H), lambda i: (0, 0))
    bspec = pl.BlockSpec((1, H), lambda i: (0, 0))
    return pl.pallas_call(
        _node_body,
        grid=(NBN,),
        in_specs=[pl.BlockSpec((BN, H), lambda i: (i, 0)),
                  pl.BlockSpec((BN, H), lambda i: (i, 0)),
                  pl.BlockSpec((2, BN, 128), lambda i: (0, i, 0)),
                  pl.BlockSpec((2, BN, 128), lambda i: (0, i, 0)),
                  bspec,
                  wspec, wspec, wspec, wspec, wspec, bspec,
                  wspec, wspec, wspec, bspec,
                  wspec, bspec,
                  pl.BlockSpec((H, 1), lambda i: (0, 0)),
                  pl.BlockSpec((1, 1), lambda i: (0, 0))],
        out_specs=[pl.BlockSpec((BN, H), lambda i: (i, 0)),
                   pl.BlockSpec((BN, 1), lambda i: (i, 0)),
                   pl.BlockSpec((1, H), lambda i: (0, 0))],
        out_shape=[jax.ShapeDtypeStruct((N, H), _f32),
                   jax.ShapeDtypeStruct((N, 1), _f32),
                   jax.ShapeDtypeStruct((1, H), _f32)],
        scratch_shapes=[pltpu.VMEM((1, H), _f32),
                        pltpu.VMEM((1, H), _f32)],
    )(ex, hx, rp, sp, hg, wn1, wn2, wn3, wn4, wn5, bnb,
      g1, g2, g3, bgb, wd1, bd1, wd2, bd2)


_SC_MESH = plsc.VectorSubcoreMesh(core_axis_name="c", subcore_axis_name="s")

NSUB = 6
NPP = 9472
ACCROWS = 9728
ACC_PT2 = ACCROWS // NS
CPT = NPP // NS
BIGIDX = 1 << 26
NPART = NSUB * NPP
NCH = EPW // CHUNK


def _sc_ids():
    cid = lax.axis_index("c")
    sid = lax.axis_index("s")
    return cid, sid, sid * NC + cid


@functools.partial(
    pl.kernel,
    mesh=_SC_MESH,
    out_type=jax.ShapeDtypeStruct((E_PAD, H), _f32),
    scratch_types=[pltpu.VMEM((CHUNK, H), _f32),
                   pltpu.VMEM((CHUNK, 128), _f32),
                   pltpu.VMEM((CHUNK, 128), _f32),
                   pltpu.VMEM((CHUNK,), jnp.int32),
                   pltpu.VMEM((CHUNK,), jnp.int32),
                   pltpu.SemaphoreType.DMA,
                   pltpu.SemaphoreType.DMA,
                   pltpu.SemaphoreType.DMA,
                   pltpu.SemaphoreType.DMA,
                   pltpu.SemaphoreType.DMA],
)
def _sc_compute(z_hbm, tw_hbm, sidxg_hbm, ridxg_hbm, newe_hbm,
                z_v, sg_v, rg_v, ia_v, ib_v, sem1, sem2, sem3, sem4, sem5):
    cid, sid, wid = _sc_ids()

    @pl.loop(0, EPW // CHUNK)
    def _chunk(k):
        e0 = pl.multiple_of(wid * EPW + k * CHUNK, 8)
        ca = pltpu.async_copy(sidxg_hbm.at[pl.ds(e0, CHUNK)], ia_v, sem3)
        cb = pltpu.async_copy(ridxg_hbm.at[pl.ds(e0, CHUNK)], ib_v, sem4)
        cz = pltpu.async_copy(z_hbm.at[pl.ds(e0, CHUNK)], z_v, sem5)
        ca.wait()
        cb.wait()
        cp1 = pltpu.async_copy(tw_hbm.at[ia_v], sg_v, sem1)
        cp2 = pltpu.async_copy(tw_hbm.at[ib_v], rg_v, sem2)
        cz.wait()
        cp1.wait()
        cp2.wait()

        def _fuse(r, carry):
            v0 = (z_v[r, pl.ds(0, 16)] + sg_v[r, pl.ds(0, 16)]
                  + rg_v[r, pl.ds(32, 16)])
            z_v[r, pl.ds(0, 16)] = jnp.maximum(v0, 0.0)
            v1 = (z_v[r, pl.ds(16, 16)] + sg_v[r, pl.ds(16, 16)]
                  + rg_v[r, pl.ds(48, 16)])
            z_v[r, pl.ds(16, 16)] = jnp.maximum(v1, 0.0)
            return carry

        lax.fori_loop(0, CHUNK, _fuse, 0, unroll=8)
        pltpu.sync_copy(z_v, newe_hbm.at[pl.ds(e0, CHUNK)])


@functools.partial(
    pl.kernel,
    mesh=_SC_MESH,
    out_type=jax.ShapeDtypeStruct((2, NPART, 128), _f32),
    scratch_types=[pltpu.VMEM((16, 128), _f32),
                   pltpu.VMEM((CHUNK, 128), _f32),
                   pltpu.VMEM((2, CHUNK, H), _f32),
                   pltpu.VMEM((2, CHUNK), jnp.int32),
                   pltpu.VMEM_SHARED((ACCROWS, 128), _f32),
                   pltpu.SemaphoreType.DMA,
                   pltpu.SemaphoreType.DMA],
)
def _sc_aggregate(newe_hbm, idx_hbm, part_hbm, zb_v, wide_v, nb_v, ia_v, acc,
                  s_l0, s_l1):
    cid, sid, wid = _sc_ids()
    zeros16 = jnp.zeros((16,), _f32)
    sems = (s_l0, s_l1)

    def _zero_rows(buf, nrows):
        def _z(r, carry):
            for h in range(8):
                buf[r, pl.ds(h * 16, 16)] = zeros16
            return carry

        lax.fori_loop(0, nrows, _z, 0, unroll=4)

    _zero_rows(zb_v, 16)
    _zero_rows(wide_v, CHUNK)

    def _e0(k):
        return pl.multiple_of(wid * EPW + k * CHUNK, 8)

    def _start_load(k, slot):
        e0 = _e0(k)
        pltpu.async_copy(idx_hbm.at[pl.ds(e0, CHUNK)], ia_v.at[slot],
                         sems[slot])
        pltpu.async_copy(newe_hbm.at[pl.ds(e0, CHUNK)], nb_v.at[slot],
                         sems[slot])

    def _wait_load(slot):
        pltpu.make_async_copy(idx_hbm.at[pl.ds(0, CHUNK)], ia_v.at[slot],
                              sems[slot]).wait()
        pltpu.make_async_copy(newe_hbm.at[pl.ds(0, CHUNK)], nb_v.at[slot],
                              sems[slot]).wait()

    for p in range(NSUB):
        base = p * NPP

        @pl.loop(0, ACC_PT2 // 16)
        def _zero_acc(k):
            pltpu.sync_copy(zb_v, acc.at[pl.ds(sid * ACC_PT2 + k * 16, 16)])

        plsc.subcore_barrier()

        def _process(slot):
            def _st(r, carry):
                wide_v[r, pl.ds(0, 16)] = nb_v[slot, r, pl.ds(0, 16)]
                wide_v[r, pl.ds(16, 16)] = nb_v[slot, r, pl.ds(16, 16)]
                return carry

            lax.fori_loop(0, CHUNK, _st, 0, unroll=8)
            for g in range(CHUNK // 16):
                raw = ia_v[slot, pl.ds(g * 16, 16)]
                t = raw - base
                ok = (t >= 0) & (t < NPP)
                ia_v[slot, pl.ds(g * 16, 16)] = jnp.where(ok, t, NPP)
            pltpu.sync_copy(wide_v, acc.at[ia_v.at[slot]], add=True)

        _start_load(0, 0)

        @pl.loop(0, NCH // 2)
        def _duo(j):
            k0 = 2 * j
            _start_load(k0 + 1, 1)
            _wait_load(0)
            _process(0)

            @pl.when(j < NCH // 2 - 1)
            def _():
                _start_load(k0 + 2, 0)

            _wait_load(1)
            _process(1)

        plsc.subcore_barrier()

        @pl.loop(0, CPT // 16)
        def _co(k):
            off = pl.multiple_of(sid * CPT + k * 16, 8)
            pltpu.sync_copy(acc.at[pl.ds(off, 16)], zb_v)
            pltpu.sync_copy(zb_v, part_hbm.at[cid, pl.ds(base + off, 16)])

        plsc.subcore_barrier()
        _zero_rows(zb_v, 16)


def kernel(node_attr, edge_index, edge_attr, global_attr, x_masks, x_holdouts,
           indicates, stage, num_processing_steps,
           W_node_enc, b_node_enc, W_edge_enc, b_edge_enc, W_glob_enc,
           b_glob_enc, W_eb, b_eb, W_nb, b_nb, W_gb, b_gb,
           W_dec1, b_dec1, W_dec2, b_dec2):
    senders = edge_index[0]
    receivers = edge_index[1]
    pad = E_PAD - E
    zpad = jnp.zeros((pad,), jnp.int32)
    dpad = jnp.full((pad,), BIGIDX, jnp.int32)
    sidx_g = jnp.concatenate([senders, zpad])
    ridx_g = jnp.concatenate([receivers, zpad])
    sidx_s = jnp.concatenate([senders, dpad])
    ridx_s = jnp.concatenate([receivers, dpad])

    A1, A2 = W_eb[0:32], W_eb[32:64]
    Ws1, Ws2 = W_eb[64:96], W_eb[96:128]
    Wr1, Wr2 = W_eb[128:160], W_eb[160:192]
    Ag = W_eb[192:224]
    A1s = A1 + A2
    Wn1, Wn2, Wn3, Wn4, Wn5 = (W_nb[0:32], W_nb[32:64], W_nb[64:96],
                               W_nb[96:128], W_nb[128:160])
    G1, G2, G3 = W_gb[0:32], W_gb[32:64], W_gb[64:96]
    beb = b_eb.reshape(1, H)
    benc = b_edge_enc.reshape(1, H)
    bnb = b_nb.reshape(1, H)
    bgb = b_gb.reshape(1, H)
    bd1 = b_dec1.reshape(1, H)
    bd2 = b_dec2.reshape(1, 1)
    wd2 = W_dec2

    x2 = node_attr.reshape(2 * N, F)
    encx = _enc_nodes(x2, W_node_enc, b_node_enc.reshape(1, H))
    ex0, ex1 = encx[0:N], encx[N:2 * N]
    encg = _glob_enc(global_attr, W_glob_enc, b_glob_enc.reshape(1, H))
    hg0 = encg[0:1]

    ea0, ea1 = edge_attr[0], edge_attr[1]

    TW0 = _prep(ex0, ex0, hg0, Ws1, Ws2, Wr1, Wr2, Ag, beb)
    Z0 = _z_step0(ea0, W_edge_enc, benc, A1s)
    newe0 = _sc_compute(Z0, TW0, sidx_g, ridx_g)
    rp0 = _sc_aggregate(newe0, ridx_s)
    sp0 = _sc_aggregate(newe0, sidx_s)
    nx0, dec0, ng0 = _node_block(ex0, ex0, rp0, sp0, hg0,
                                 Wn1, Wn2, Wn3, Wn4, Wn5, bnb,
                                 G1, G2, G3, bgb, W_dec1, bd1, wd2, bd2)

    TW1 = _prep(ex1, nx0, ng0, Ws1, Ws2, Wr1, Wr2, Ag, beb)
    Z1 = _z_step1(ea1, newe0, W_edge_enc, benc, A1, A2)
    newe1 = _sc_compute(Z1, TW1, sidx_g, ridx_g)
    rp1 = _sc_aggregate(newe1, ridx_s)
    sp1 = _sc_aggregate(newe1, sidx_s)
    nx1, dec1, ng1 = _node_block(ex1, nx0, rp1, sp1, ng0,
                                 Wn1, Wn2, Wn3, Wn4, Wn5, bnb,
                                 G1, G2, G3, bgb, W_dec1, bd1, wd2, bd2)

    return jnp.stack([dec0, dec1])


# speedup vs baseline: 1.1337x; 1.0178x over previous
import functools

import jax
import jax.numpy as jnp
from jax import lax
from jax.experimental import pallas as pl
from jax.experimental.pallas import tpu as pltpu
from jax.experimental.pallas import tpu_sc as plsc

N = 50000
E = 800000
H = 32
F = 64

NC = 2
NS = 16
NW = NC * NS
EPW = 25600
E_PAD = NW * EPW
CHUNK = 128

BN = 2000
NBN = N // BN
BE = 4000
NBE = E // BE

_f32 = jnp.float32


def _dot(a, b):
    return jnp.dot(a, b, preferred_element_type=_f32)


def _enc_body(x_ref, w_ref, b_ref, o_ref):
    o_ref[...] = jnp.maximum(_dot(x_ref[...], w_ref[...]) + b_ref[...], 0.0)


def _enc_nodes(x2, w, b):
    m = x2.shape[0]
    return pl.pallas_call(
        _enc_body,
        grid=(m // BN,),
        in_specs=[pl.BlockSpec((BN, F), lambda i: (i, 0)),
                  pl.BlockSpec((F, H), lambda i: (0, 0)),
                  pl.BlockSpec((1, H), lambda i: (0, 0))],
        out_specs=pl.BlockSpec((BN, H), lambda i: (i, 0)),
        out_shape=jax.ShapeDtypeStruct((m, H), _f32),
    )(x2, w, b)


def _glob_enc(g, w, b):
    return pl.pallas_call(
        _enc_body,
        out_shape=jax.ShapeDtypeStruct((2, H), _f32),
    )(g, w, b)


def _prep_body(ex_ref, hx_ref, hg_ref, ws1, ws2, wr1, wr2, ag, beb, tw_ref):
    ce = _dot(hg_ref[...], ag[...]) + beb[...]
    s = _dot(ex_ref[...], ws1[...]) + _dot(hx_ref[...], ws2[...]) + ce
    r = _dot(ex_ref[...], wr1[...]) + _dot(hx_ref[...], wr2[...])
    tw_ref[...] = jnp.concatenate([s, r, s, r], axis=1)


def _prep(ex, hx, hg, ws1, ws2, wr1, wr2, ag, beb):
    wspec = pl.BlockSpec((H, H), lambda i: (0, 0))
    return pl.pallas_call(
        _prep_body,
        grid=(NBN,),
        in_specs=[pl.BlockSpec((BN, H), lambda i: (i, 0)),
                  pl.BlockSpec((BN, H), lambda i: (i, 0)),
                  pl.BlockSpec((1, H), lambda i: (0, 0)),
                  wspec, wspec, wspec, wspec, wspec,
                  pl.BlockSpec((1, H), lambda i: (0, 0))],
        out_specs=pl.BlockSpec((BN, 128), lambda i: (i, 0)),
        out_shape=jax.ShapeDtypeStruct((N, 128), _f32),
    )(ex, hx, hg, ws1, ws2, wr1, wr2, ag, beb)


def _z0_body(ea_ref, wenc, benc, a1s, z_ref):
    e = jnp.maximum(ea_ref[...] * wenc[...] + benc[...], 0.0)
    z_ref[...] = _dot(e, a1s[...])


def _z_step0(ea, wenc, benc, a1s):
    return pl.pallas_call(
        _z0_body,
        grid=(NBE,),
        in_specs=[pl.BlockSpec((BE, 1), lambda i: (i, 0)),
                  pl.BlockSpec((1, H), lambda i: (0, 0)),
                  pl.BlockSpec((1, H), lambda i: (0, 0)),
                  pl.BlockSpec((H, H), lambda i: (0, 0))],
        out_specs=pl.BlockSpec((BE, H), lambda i: (i, 0)),
        out_shape=jax.ShapeDtypeStruct((E_PAD, H), _f32),
    )(ea, wenc, benc, a1s)


def _z1_body(ea_ref, he_ref, wenc, benc, a1, a2, z_ref):
    e = jnp.maximum(ea_ref[...] * wenc[...] + benc[...], 0.0)
    z_ref[...] = _dot(e, a1[...]) + _dot(he_ref[...], a2[...])


def _z_step1(ea, he, wenc, benc, a1, a2):
    return pl.pallas_call(
        _z1_body,
        grid=(NBE,),
        in_specs=[pl.BlockSpec((BE, 1), lambda i: (i, 0)),
                  pl.BlockSpec((BE, H), lambda i: (i, 0)),
                  pl.BlockSpec((1, H), lambda i: (0, 0)),
                  pl.BlockSpec((1, H), lambda i: (0, 0)),
                  pl.BlockSpec((H, H), lambda i: (0, 0)),
                  pl.BlockSpec((H, H), lambda i: (0, 0))],
        out_specs=pl.BlockSpec((BE, H), lambda i: (i, 0)),
        out_shape=jax.ShapeDtypeStruct((E_PAD, H), _f32),
    )(ea, he, wenc, benc, a1, a2)


def _node_body(ex_ref, hx_ref, rp_ref, sp_ref, hg_ref,
               wn1, wn2, wn3, wn4, wn5, bnb,
               g1, g2, g3, bgb, wd1, bd1, wd2, bd2,
               nx_ref, dec_ref, ng_ref, sx_acc, se_acc):
    i = pl.program_id(0)
    ra = rp_ref[0, :, 0:H] + rp_ref[1, :, 0:H]
    sa = sp_ref[0, :, 0:H] + sp_ref[1, :, 0:H]
    cn = _dot(hg_ref[...], wn5[...]) + bnb[...]
    nx = jnp.maximum(_dot(ex_ref[...], wn1[...]) + _dot(hx_ref[...], wn2[...])
                     + _dot(ra, wn3[...]) + _dot(sa, wn4[...]) + cn, 0.0)
    nx_ref[...] = nx
    d1 = jnp.maximum(_dot(nx, wd1[...]) + bd1[...], 0.0)
    dec_ref[...] = _dot(d1, wd2[...]) + bd2[...]
    sx = jnp.sum(nx, axis=0, keepdims=True)
    se = jnp.sum(ra, axis=0, keepdims=True)

    @pl.when(i == 0)
    def _():
        sx_acc[...] = sx
        se_acc[...] = se

    @pl.when(i > 0)
    def _():
        sx_acc[...] = sx_acc[...] + sx
        se_acc[...] = se_acc[...] + se

    @pl.when(i == NBN - 1)
    def _():
        mx = sx_acc[...] * (1.0 / N)
        me = se_acc[...] * (1.0 / E)
        ng_ref[...] = jnp.maximum(_dot(mx, g1[...]) + _dot(me, g2[...])
                                  + _dot(hg_ref[...], g3[...]) + bgb[...], 0.0)


def _node_block(ex, hx, rp, sp, hg, wn1, wn2, wn3, wn4, wn5, bnb,
                g1, g2, g3, bgb, wd1, bd1, wd2, bd2):
    wspec = pl.BlockSpec((H, H), lambda i: (0, 0))
    bspec = pl.BlockSpec((1, H), lambda i: (0, 0))
    return pl.pallas_call(
        _node_body,
        grid=(NBN,),
        in_specs=[pl.BlockSpec((BN, H), lambda i: (i, 0)),
                  pl.BlockSpec((BN, H), lambda i: (i, 0)),
                  pl.BlockSpec((2, BN, 128), lambda i: (0, i, 0)),
                  pl.BlockSpec((2, BN, 128), lambda i: (0, i, 0)),
                  bspec,
                  wspec, wspec, wspec, wspec, wspec, bspec,
                  wspec, wspec, wspec, bspec,
                  wspec, bspec,
                  pl.BlockSpec((H, 1), lambda i: (0, 0)),
                  pl.BlockSpec((1, 1), lambda i: (0, 0))],
        out_specs=[pl.BlockSpec((BN, H), lambda i: (i, 0)),
                   pl.BlockSpec((BN, 1), lambda i: (i, 0)),
                   pl.BlockSpec((1, H), lambda i: (0, 0))],
        out_shape=[jax.ShapeDtypeStruct((N, H), _f32),
                   jax.ShapeDtypeStruct((N, 1), _f32),
                   jax.ShapeDtypeStruct((1, H), _f32)],
        scratch_shapes=[pltpu.VMEM((1, H), _f32),
                        pltpu.VMEM((1, H), _f32)],
    )(ex, hx, rp, sp, hg, wn1, wn2, wn3, wn4, wn5, bnb,
      g1, g2, g3, bgb, wd1, bd1, wd2, bd2)


_SC_MESH = plsc.VectorSubcoreMesh(core_axis_name="c", subcore_axis_name="s")

NSUB = 6
NPP = 9472
ACCROWS = 9728
ACC_PT2 = ACCROWS // NS
CPT = NPP // NS
BIGIDX = 1 << 26
NPART = NSUB * NPP
NCH = EPW // CHUNK


def _sc_ids():
    cid = lax.axis_index("c")
    sid = lax.axis_index("s")
    return cid, sid, sid * NC + cid


@functools.partial(
    pl.kernel,
    mesh=_SC_MESH,
    out_type=jax.ShapeDtypeStruct((E_PAD, H), _f32),
    scratch_types=[pltpu.VMEM((2, CHUNK, H), _f32),
                   pltpu.VMEM((2, CHUNK, 128), _f32),
                   pltpu.VMEM((2, CHUNK, 128), _f32),
                   pltpu.VMEM((2, CHUNK), jnp.int32),
                   pltpu.VMEM((2, CHUNK), jnp.int32),
                   pltpu.SemaphoreType.DMA,
                   pltpu.SemaphoreType.DMA,
                   pltpu.SemaphoreType.DMA,
                   pltpu.SemaphoreType.DMA],
)
def _sc_compute(z_hbm, tw_hbm, sidxg_hbm, ridxg_hbm, newe_hbm,
                z_v, sg_v, rg_v, ia_v, ib_v, s_i0, s_i1, s_g0, s_g1):
    cid, sid, wid = _sc_ids()
    sems_i = (s_i0, s_i1)
    sems_g = (s_g0, s_g1)

    def _e0(k):
        return pl.multiple_of(wid * EPW + k * CHUNK, 8)

    def _start_idxz(k, slot):
        e0 = _e0(k)
        pltpu.async_copy(sidxg_hbm.at[pl.ds(e0, CHUNK)], ia_v.at[slot],
                         sems_i[slot])
        pltpu.async_copy(ridxg_hbm.at[pl.ds(e0, CHUNK)], ib_v.at[slot],
                         sems_i[slot])
        pltpu.async_copy(z_hbm.at[pl.ds(e0, CHUNK)], z_v.at[slot],
                         sems_i[slot])

    def _wait_idxz(slot):
        pltpu.make_async_copy(sidxg_hbm.at[pl.ds(0, CHUNK)], ia_v.at[slot],
                              sems_i[slot]).wait()
        pltpu.make_async_copy(ridxg_hbm.at[pl.ds(0, CHUNK)], ib_v.at[slot],
                              sems_i[slot]).wait()
        pltpu.make_async_copy(z_hbm.at[pl.ds(0, CHUNK)], z_v.at[slot],
                              sems_i[slot]).wait()

    def _start_gathers(slot):
        pltpu.async_copy(tw_hbm.at[ia_v.at[slot]], sg_v.at[slot], sems_g[slot])
        pltpu.async_copy(tw_hbm.at[ib_v.at[slot]], rg_v.at[slot], sems_g[slot])

    def _wait_gathers(slot):
        pltpu.make_async_copy(tw_hbm.at[ia_v.at[slot]], sg_v.at[slot],
                              sems_g[slot]).wait()
        pltpu.make_async_copy(tw_hbm.at[ib_v.at[slot]], rg_v.at[slot],
                              sems_g[slot]).wait()

    def _fuse_write(k, slot):
        def _fuse(r, carry):
            v0 = (z_v[slot, r, pl.ds(0, 16)] + sg_v[slot, r, pl.ds(0, 16)]
                  + rg_v[slot, r, pl.ds(32, 16)])
            z_v[slot, r, pl.ds(0, 16)] = jnp.maximum(v0, 0.0)
            v1 = (z_v[slot, r, pl.ds(16, 16)] + sg_v[slot, r, pl.ds(16, 16)]
                  + rg_v[slot, r, pl.ds(48, 16)])
            z_v[slot, r, pl.ds(16, 16)] = jnp.maximum(v1, 0.0)
            return carry

        lax.fori_loop(0, CHUNK, _fuse, 0, unroll=8)
        pltpu.sync_copy(z_v.at[slot], newe_hbm.at[pl.ds(_e0(k), CHUNK)])

    _start_idxz(0, 0)
    _wait_idxz(0)
    _start_gathers(0)

    @pl.loop(0, NCH // 2)
    def _duo(j):
        k0 = 2 * j
        _start_idxz(k0 + 1, 1)
        _wait_gathers(0)
        _fuse_write(k0, 0)
        _wait_idxz(1)
        _start_gathers(1)

        @pl.when(j < NCH // 2 - 1)
        def _():
            _start_idxz(k0 + 2, 0)

        _wait_gathers(1)
        _fuse_write(k0 + 1, 1)

        @pl.when(j < NCH // 2 - 1)
        def _():
            _wait_idxz(0)
            _start_gathers(0)


@functools.partial(
    pl.kernel,
    mesh=_SC_MESH,
    out_type=jax.ShapeDtypeStruct((2, NPART, 128), _f32),
    scratch_types=[pltpu.VMEM((16, 128), _f32),
                   pltpu.VMEM((CHUNK, 128), _f32),
                   pltpu.VMEM((2, CHUNK, H), _f32),
                   pltpu.VMEM((2, CHUNK), jnp.int32),
                   pltpu.VMEM_SHARED((ACCROWS, 128), _f32),
                   pltpu.SemaphoreType.DMA,
                   pltpu.SemaphoreType.DMA],
)
def _sc_aggregate(newe_hbm, idx_hbm, part_hbm, zb_v, wide_v, nb_v, ia_v, acc,
                  s_l0, s_l1):
    cid, sid, wid = _sc_ids()
    zeros16 = jnp.zeros((16,), _f32)
    sems = (s_l0, s_l1)

    def _zero_rows(buf, nrows):
        def _z(r, carry):
            for h in range(8):
                buf[r, pl.ds(h * 16, 16)] = zeros16
            return carry

        lax.fori_loop(0, nrows, _z, 0, unroll=4)

    _zero_rows(zb_v, 16)
    _zero_rows(wide_v, CHUNK)

    def _e0(k):
        return pl.multiple_of(wid * EPW + k * CHUNK, 8)

    def _start_load(k, slot):
        e0 = _e0(k)
        pltpu.async_copy(idx_hbm.at[pl.ds(e0, CHUNK)], ia_v.at[slot],
                         sems[slot])
        pltpu.async_copy(newe_hbm.at[pl.ds(e0, CHUNK)], nb_v.at[slot],
                         sems[slot])

    def _wait_load(slot):
        pltpu.make_async_copy(idx_hbm.at[pl.ds(0, CHUNK)], ia_v.at[slot],
                              sems[slot]).wait()
        pltpu.make_async_copy(newe_hbm.at[pl.ds(0, CHUNK)], nb_v.at[slot],
                              sems[slot]).wait()

    for p in range(NSUB):
        base = p * NPP

        @pl.loop(0, ACC_PT2 // 16)
        def _zero_acc(k):
            pltpu.sync_copy(zb_v, acc.at[pl.ds(sid * ACC_PT2 + k * 16, 16)])

        plsc.subcore_barrier()

        def _process(slot):
            def _st(r, carry):
                wide_v[r, pl.ds(0, 16)] = nb_v[slot, r, pl.ds(0, 16)]
                wide_v[r, pl.ds(16, 16)] = nb_v[slot, r, pl.ds(16, 16)]
                return carry

            lax.fori_loop(0, CHUNK, _st, 0, unroll=8)
            for g in range(CHUNK // 16):
                raw = ia_v[slot, pl.ds(g * 16, 16)]
                t = raw - base
                ok = (t >= 0) & (t < NPP)
                ia_v[slot, pl.ds(g * 16, 16)] = jnp.where(ok, t, NPP)
            pltpu.sync_copy(wide_v, acc.at[ia_v.at[slot]], add=True)

        _start_load(0, 0)

        @pl.loop(0, NCH // 2)
        def _duo(j):
            k0 = 2 * j
            _start_load(k0 + 1, 1)
            _wait_load(0)
            _process(0)

            @pl.when(j < NCH // 2 - 1)
            def _():
                _start_load(k0 + 2, 0)

            _wait_load(1)
            _process(1)

        plsc.subcore_barrier()

        @pl.loop(0, CPT // 16)
        def _co(k):
            off = pl.multiple_of(sid * CPT + k * 16, 8)
            pltpu.sync_copy(acc.at[pl.ds(off, 16)], zb_v)
            pltpu.sync_copy(zb_v, part_hbm.at[cid, pl.ds(base + off, 16)])

        plsc.subcore_barrier()
        _zero_rows(zb_v, 16)


def kernel(node_attr, edge_index, edge_attr, global_attr, x_masks, x_holdouts,
           indicates, stage, num_processing_steps,
           W_node_enc, b_node_enc, W_edge_enc, b_edge_enc, W_glob_enc,
           b_glob_enc, W_eb, b_eb, W_nb, b_nb, W_gb, b_gb,
           W_dec1, b_dec1, W_dec2, b_dec2):
    senders = edge_index[0]
    receivers = edge_index[1]
    pad = E_PAD - E
    zpad = jnp.zeros((pad,), jnp.int32)
    dpad = jnp.full((pad,), BIGIDX, jnp.int32)
    sidx_g = jnp.concatenate([senders, zpad])
    ridx_g = jnp.concatenate([receivers, zpad])
    sidx_s = jnp.concatenate([senders, dpad])
    ridx_s = jnp.concatenate([receivers, dpad])

    A1, A2 = W_eb[0:32], W_eb[32:64]
    Ws1, Ws2 = W_eb[64:96], W_eb[96:128]
    Wr1, Wr2 = W_eb[128:160], W_eb[160:192]
    Ag = W_eb[192:224]
    A1s = A1 + A2
    Wn1, Wn2, Wn3, Wn4, Wn5 = (W_nb[0:32], W_nb[32:64], W_nb[64:96],
                               W_nb[96:128], W_nb[128:160])
    G1, G2, G3 = W_gb[0:32], W_gb[32:64], W_gb[64:96]
    beb = b_eb.reshape(1, H)
    benc = b_edge_enc.reshape(1, H)
    bnb = b_nb.reshape(1, H)
    bgb = b_gb.reshape(1, H)
    bd1 = b_dec1.reshape(1, H)
    bd2 = b_dec2.reshape(1, 1)
    wd2 = W_dec2

    x2 = node_attr.reshape(2 * N, F)
    encx = _enc_nodes(x2, W_node_enc, b_node_enc.reshape(1, H))
    ex0, ex1 = encx[0:N], encx[N:2 * N]
    encg = _glob_enc(global_attr, W_glob_enc, b_glob_enc.reshape(1, H))
    hg0 = encg[0:1]

    ea0, ea1 = edge_attr[0], edge_attr[1]

    TW0 = _prep(ex0, ex0, hg0, Ws1, Ws2, Wr1, Wr2, Ag, beb)
    Z0 = _z_step0(ea0, W_edge_enc, benc, A1s)
    newe0 = _sc_compute(Z0, TW0, sidx_g, ridx_g)
    rp0 = _sc_aggregate(newe0, ridx_s)
    sp0 = _sc_aggregate(newe0, sidx_s)
    nx0, dec0, ng0 = _node_block(ex0, ex0, rp0, sp0, hg0,
                                 Wn1, Wn2, Wn3, Wn4, Wn5, bnb,
                                 G1, G2, G3, bgb, W_dec1, bd1, wd2, bd2)

    TW1 = _prep(ex1, nx0, ng0, Ws1, Ws2, Wr1, Wr2, Ag, beb)
    Z1 = _z_step1(ea1, newe0, W_edge_enc, benc, A1, A2)
    newe1 = _sc_compute(Z1, TW1, sidx_g, ridx_g)
    rp1 = _sc_aggregate(newe1, ridx_s)
    sp1 = _sc_aggregate(newe1, sidx_s)
    nx1, dec1, ng1 = _node_block(ex1, nx0, rp1, sp1, ng0,
                                 Wn1, Wn2, Wn3, Wn4, Wn5, bnb,
                                 G1, G2, G3, bgb, W_dec1, bd1, wd2, bd2)

    return jnp.stack([dec0, dec1])
